# baseline (speedup 1.0000x reference)
import functools
import jax
import jax.numpy as jnp
from jax import lax
from jax.experimental import pallas as pl
from jax.experimental.pallas import tpu as pltpu
from jax.experimental.pallas import tpu_sc as plsc

D_MODEL = 64
SCALE = 8.0

NC = 2
NS = 16
NW = NC * NS
LANES = 16

GATHER = 128


def _emb_body(ng, x_hbm, table_hbm, out_hbm, idx_v, rows_v, gsem):
    wid = lax.axis_index("s") * NC + lax.axis_index("c")
    pltpu.sync_copy(x_hbm.at[wid], idx_v)

    def g_body(g, _):
        pltpu.async_copy(table_hbm.at[idx_v.at[g]], rows_v, gsem).wait()

        def r_body(r, _):
            for k in range(D_MODEL // LANES):
                sl = pl.ds(k * LANES, LANES)
                rows_v[r, sl] = rows_v[r, sl] * SCALE
            return ()

        lax.fori_loop(0, GATHER, r_body, ())
        out_base = (wid * ng + g) * GATHER
        pltpu.sync_copy(rows_v, out_hbm.at[pl.ds(out_base, GATHER)])
        return ()

    lax.fori_loop(0, ng, g_body, ())


def kernel(x, table):
    orig_shape = x.shape
    b = x.size
    assert b % (NW * GATHER) == 0
    ng = b // (NW * GATHER)

    x_flat = x.reshape(NW, ng, GATHER).astype(jnp.int32)

    mesh = plsc.VectorSubcoreMesh(
        core_axis_name="c", subcore_axis_name="s", num_cores=NC, num_subcores=NS
    )
    run = pl.kernel(
        functools.partial(_emb_body, ng),
        out_type=jax.ShapeDtypeStruct((b, D_MODEL), jnp.float32),
        mesh=mesh,
        scratch_types=[
            pltpu.VMEM((ng, GATHER), jnp.int32),
            pltpu.VMEM((GATHER, D_MODEL), jnp.float32),
            pltpu.SemaphoreType.DMA,
        ],
        compiler_params=pltpu.CompilerParams(use_tc_tiling_on_sc=False),
    )
    out = run(x_flat, table)
    return out.reshape(*orig_shape, D_MODEL)


# speedup vs baseline: 1.2048x; 1.2048x over previous
import functools
import jax
import jax.numpy as jnp
from jax import lax
from jax.experimental import pallas as pl
from jax.experimental.pallas import tpu as pltpu
from jax.experimental.pallas import tpu_sc as plsc

D_MODEL = 64
SCALE = 8.0

NC = 2
NS = 16
NW = NC * NS
LANES = 16

GATHER = 128
NBUF = 4


def _scale_buf(buf):
    @plsc.parallel_loop(0, GATHER, unroll=4)
    def _(r):
        for k in range(D_MODEL // LANES):
            sl = pl.ds(k * LANES, LANES)
            buf[r, sl] = buf[r, sl] * SCALE


def _emb_body(ng, x_hbm, table_hbm, out_hbm, idx_v, *bufs_and_sems):
    rows = bufs_and_sems[:NBUF]
    gsem = bufs_and_sems[NBUF:2 * NBUF]
    ssem = bufs_and_sems[2 * NBUF:3 * NBUF]

    wid = lax.axis_index("s") * NC + lax.axis_index("c")
    out0 = wid * ng * GATHER
    pltpu.sync_copy(x_hbm.at[wid], idx_v)

    def fire_gather(g, b):
        return pltpu.async_copy(table_hbm.at[idx_v.at[g]], rows[b], gsem[b])

    def fire_scatter(g, b):
        return pltpu.async_copy(
            rows[b], out_hbm.at[pl.ds(out0 + g * GATHER, GATHER)], ssem[b])

    def process(g, b):
        pltpu.make_async_copy(table_hbm.at[idx_v.at[g]], rows[b], gsem[b]).wait()
        _scale_buf(rows[b])
        fire_scatter(g, b)

    fire_gather(0, 0)
    fire_gather(1, 1)
    for g in (0, 1):
        fire_gather(g + 2, (g + 2) % NBUF)
        process(g, g % NBUF)

    def outer(go, _):
        g0 = 2 + go * NBUF
        for j in range(NBUF):
            g = g0 + j
            b = (2 + j) % NBUF
            bn = (b + 2) % NBUF
            pltpu.make_async_copy(
                rows[bn], out_hbm.at[pl.ds(out0, GATHER)], ssem[bn]).wait()
            fire_gather(g + 2, bn)
            process(g, b)
        return ()

    lax.fori_loop(0, (ng - 4) // NBUF, outer, ())

    for g in (ng - 2, ng - 1):
        process(g, g % NBUF)
    for b in range(NBUF):
        pltpu.make_async_copy(
            rows[b], out_hbm.at[pl.ds(out0, GATHER)], ssem[b]).wait()


def kernel(x, table):
    orig_shape = x.shape
    b = x.size
    assert b % (NW * GATHER) == 0
    ng = b // (NW * GATHER)
    assert (ng - 4) % NBUF == 0

    x_flat = x.reshape(NW, ng, GATHER).astype(jnp.int32)

    mesh = plsc.VectorSubcoreMesh(
        core_axis_name="c", subcore_axis_name="s", num_cores=NC, num_subcores=NS
    )
    run = pl.kernel(
        functools.partial(_emb_body, ng),
        out_type=jax.ShapeDtypeStruct((b, D_MODEL), jnp.float32),
        mesh=mesh,
        scratch_types=(
            [pltpu.VMEM((ng, GATHER), jnp.int32)]
            + [pltpu.VMEM((GATHER, D_MODEL), jnp.float32) for _ in range(NBUF)]
            + [pltpu.SemaphoreType.DMA for _ in range(2 * NBUF)]
        ),
        compiler_params=pltpu.CompilerParams(use_tc_tiling_on_sc=False),
    )
    out = run(x_flat, table)
    return out.reshape(*orig_shape, D_MODEL)
